# baseline (speedup 1.0000x reference)
import functools

import jax
import jax.numpy as jnp
from jax import lax
from jax.experimental import pallas as pl
from jax.experimental.pallas import tpu as pltpu
from jax.experimental.pallas import tpu_sc as plsc

_NW = 32
_CW = 128
_L = 16


def kernel(embeddings, labels, centers):
    B, D = embeddings.shape
    ch = B // (_NW * _CW)
    bw = _CW * ch
    ng = _CW // _L

    emb_t = embeddings.T
    ctr2 = centers.reshape(centers.shape[0] // 2, 2 * D)
    labels = labels.astype(jnp.int32)
    pairs = labels // 2
    par64 = (labels % 2) * D

    mesh = plsc.VectorSubcoreMesh(core_axis_name="c", subcore_axis_name="s")

    @functools.partial(
        pl.kernel,
        mesh=mesh,
        compiler_params=pltpu.CompilerParams(
            use_tc_tiling_on_sc=True, needs_layout_passes=False
        ),
        out_type=jax.ShapeDtypeStruct((_NW * _L,), jnp.float32),
        scratch_types=[
            pltpu.VMEM((bw,), jnp.int32),
            pltpu.VMEM((bw,), jnp.int32),
            pltpu.VMEM((D, bw), jnp.float32),
            pltpu.VMEM((bw, 2 * D), jnp.float32),
            pltpu.VMEM((_L,), jnp.float32),
            pltpu.SemaphoreType.DMA,
            pltpu.SemaphoreType.DMA,
            pltpu.SemaphoreType.DMA,
            pltpu.SemaphoreType.DMA,
            pltpu.SemaphoreType.DMA,
            pltpu.SemaphoreType.DMA,
        ],
    )
    def sc_kernel(emb_hbm, pairs_hbm, par_hbm, ctr_hbm, out_hbm,
                  idx_v, par_v, emb_v, ctr_v, acc_v,
                  sem_e, sem_p, sem_g0, sem_g1, sem_g2, sem_g3):
        wid = lax.axis_index("s") * 2 + lax.axis_index("c")
        base = wid * bw

        emb_dma = pltpu.async_copy(emb_hbm.at[:, pl.ds(base, bw)], emb_v, sem_e)
        par_dma = pltpu.async_copy(par_hbm.at[pl.ds(base, bw)], par_v, sem_p)
        pltpu.sync_copy(pairs_hbm.at[pl.ds(base, bw)], idx_v)
        sems = [sem_g0, sem_g1, sem_g2, sem_g3]
        gathers = [
            pltpu.async_copy(
                ctr_hbm.at[idx_v.at[pl.ds(j * _CW, _CW)]],
                ctr_v.at[pl.ds(j * _CW, _CW)],
                sems[j],
            )
            for j in range(ch)
        ]
        emb_dma.wait()
        par_dma.wait()

        iota = lax.iota(jnp.int32, _L)

        def group_body(g, acc):
            rows = g * _L + iota
            cols0 = par_v[pl.ds(g * _L, _L)]
            for f in range(D):
                c = plsc.load_gather(ctr_v, [rows, cols0 + f])
                e = emb_v[f, pl.ds(g * _L, _L)]
                d = e - c
                acc = acc + d * d
            return acc

        acc = jnp.zeros((_L,), jnp.float32)
        for j in range(ch):
            gathers[j].wait()
            acc = lax.fori_loop(j * ng, (j + 1) * ng, group_body, acc)

        acc_v[...] = acc
        pltpu.sync_copy(acc_v, out_hbm.at[pl.ds(wid * _L, _L)])

    partials = sc_kernel(emb_t, pairs, par64, ctr2)
    return partials.sum() / B


# speedup vs baseline: 1.1500x; 1.1500x over previous
import functools

import jax
import jax.numpy as jnp
from jax import lax
from jax.experimental import pallas as pl
from jax.experimental.pallas import tpu as pltpu
from jax.experimental.pallas import tpu_sc as plsc

_NW = 32
_CW = 128
_L = 16


def kernel(embeddings, labels, centers):
    B, D = embeddings.shape
    ch = B // (_NW * _CW)
    bw = _CW * ch
    ng = _CW // _L
    DP = 2 * D

    emb_t = embeddings.T
    ctr_p = jnp.pad(centers, ((0, 0), (0, DP - D)))
    idx = labels.astype(jnp.int32)

    mesh = plsc.VectorSubcoreMesh(core_axis_name="c", subcore_axis_name="s")

    @functools.partial(
        pl.kernel,
        mesh=mesh,
        compiler_params=pltpu.CompilerParams(
            use_tc_tiling_on_sc=True, needs_layout_passes=False
        ),
        out_type=jax.ShapeDtypeStruct((_NW * _L,), jnp.float32),
        scratch_types=[
            pltpu.VMEM((bw,), jnp.int32),
            pltpu.VMEM((D, bw), jnp.float32),
            pltpu.VMEM((bw, DP), jnp.float32),
            pltpu.VMEM((_L,), jnp.float32),
            pltpu.SemaphoreType.DMA,
            pltpu.SemaphoreType.DMA,
            pltpu.SemaphoreType.DMA,
            pltpu.SemaphoreType.DMA,
            pltpu.SemaphoreType.DMA,
        ],
    )
    def sc_kernel(emb_hbm, idx_hbm, ctr_hbm, out_hbm,
                  idx_v, emb_v, ctr_v, acc_v,
                  sem_e, sem_g0, sem_g1, sem_g2, sem_g3):
        wid = lax.axis_index("s") * 2 + lax.axis_index("c")
        base = wid * bw

        emb_dma = pltpu.async_copy(emb_hbm.at[:, pl.ds(base, bw)], emb_v, sem_e)
        pltpu.sync_copy(idx_hbm.at[pl.ds(base, bw)], idx_v)
        sems = [sem_g0, sem_g1, sem_g2, sem_g3]
        gathers = [
            pltpu.async_copy(
                ctr_hbm.at[idx_v.at[pl.ds(j * _CW, _CW)]],
                ctr_v.at[pl.ds(j * _CW, _CW)],
                sems[j],
            )
            for j in range(ch)
        ]
        emb_dma.wait()

        iota = lax.iota(jnp.int32, _L)
        dmask = jnp.full((_L,), D - 1, jnp.int32)

        def group_body(g, acc):
            rows = g * _L + iota
            ctr_base = rows * DP
            emb_off = rows
            f_vec = iota
            for _ in range(D):
                c = plsc.load_gather(ctr_v, [rows, f_vec])
                e = plsc.load_gather(emb_v, [f_vec, emb_off])
                d = e - c
                acc = acc + d * d
                f_vec = (f_vec + 1) & dmask
            del ctr_base
            return acc

        acc = jnp.zeros((_L,), jnp.float32)
        for j in range(ch):
            gathers[j].wait()
            acc = lax.fori_loop(j * ng, (j + 1) * ng, group_body, acc)

        acc_v[...] = acc
        pltpu.sync_copy(acc_v, out_hbm.at[pl.ds(wid * _L, _L)])

    partials = sc_kernel(emb_t, idx, ctr_p)
    return partials.sum() / B
